# baseline (speedup 1.0000x reference)
import functools
import math

import jax
import jax.numpy as jnp
from jax import lax
from jax.experimental import pallas as pl
from jax.experimental.pallas import tpu as pltpu
from jax.experimental.pallas import tpu_sc as plsc

EMB = 64
NC = 2
NS = 16
NW = NC * NS
LANES = 16
CHUNK = 128
SCALE = math.sqrt(EMB)


def kernel(tokens, weight):
    batch, hist = tokens.shape
    b_per_w = batch // NW
    n_bchunk = b_per_w // CHUNK
    assert b_per_w * NW == batch and n_bchunk * CHUNK == b_per_w
    eb = EMB // 8
    bb = batch // CHUNK

    tokens_t = tokens.T.astype(jnp.int32)

    mesh = plsc.VectorSubcoreMesh(core_axis_name="c", subcore_axis_name="s")

    @functools.partial(
        pl.kernel,
        mesh=mesh,
        out_type=jax.ShapeDtypeStruct((hist, eb, bb, 8, CHUNK), jnp.float32),
        scratch_types=[
            pltpu.VMEM((hist, b_per_w), jnp.int32),
            pltpu.VMEM((n_bchunk, CHUNK, EMB), jnp.float32),
            pltpu.VMEM((2, eb, n_bchunk, 8, CHUNK), jnp.float32),
            pltpu.SemaphoreType.DMA((n_bchunk,)),
            pltpu.SemaphoreType.DMA,
            pltpu.SemaphoreType.DMA,
        ],
        compiler_params=pltpu.CompilerParams(use_tc_tiling_on_sc=False,
                                             needs_layout_passes=False),
    )
    def emb_kernel(tok_hbm, table_hbm, out_hbm, tok_v, rows_v, stage_v,
                   gsem, osem0, osem1):
        wid = lax.axis_index("s") * NC + lax.axis_index("c")
        base_b = wid * b_per_w
        pltpu.sync_copy(tok_hbm.at[:, pl.ds(base_b, b_per_w)], tok_v)

        iota = lax.iota(jnp.int32, LANES)
        e_in_idx = iota % 8
        eb_half = iota // 8

        def gather(h, c):
            pltpu.async_copy(
                table_hbm.at[tok_v.at[h, pl.ds(CHUNK * c, CHUNK)]],
                rows_v.at[c], gsem.at[c])

        def wait_gather(h, c):
            pltpu.make_async_copy(
                table_hbm.at[tok_v.at[h, pl.ds(CHUNK * c, CHUNK)]],
                rows_v.at[c], gsem.at[c]).wait()

        def out_dst(h):
            return out_hbm.at[h, :, pl.ds(n_bchunk * wid, n_bchunk)]

        def write_out(h, par, sem):
            pltpu.async_copy(stage_v.at[par], out_dst(h), sem)

        def wait_out(h, par, sem):
            pltpu.make_async_copy(stage_v.at[par], out_dst(h), sem).wait()

        def transpose_chunk(h2, c):
            src = rows_v.at[c]
            dst = stage_v.at[h2]
            c_vec = jnp.full((LANES,), c, jnp.int32)

            @plsc.parallel_loop(0, CHUNK, unroll=6)
            def rbody(r):
                b_vec = lax.broadcast(r, (LANES,))
                for m in range(EMB // LANES):
                    vec = src[r, pl.ds(LANES * m, LANES)]
                    plsc.store_scatter(
                        dst, [eb_half + 2 * m, c_vec, e_in_idx, b_vec],
                        vec * SCALE)

        for c in range(n_bchunk):
            gather(0, c)

        def hbody(h, carry):
            h2 = h % 2
            even = h2 == 0

            @pl.when((h >= 2) & even)
            def _():
                wait_out(h, 0, osem0)

            @pl.when((h >= 2) & jnp.logical_not(even))
            def _():
                wait_out(h, 1, osem1)

            for c in range(n_bchunk):
                wait_gather(h, c)
                transpose_chunk(h2, c)

                @pl.when(h < hist - 1)
                def _():
                    gather(h + 1, c)

            @pl.when(even)
            def _():
                write_out(h, 0, osem0)

            @pl.when(jnp.logical_not(even))
            def _():
                write_out(h, 1, osem1)

            return carry

        lax.fori_loop(0, hist, hbody, 0)
        wait_out(hist - 2, 0, osem0)
        wait_out(hist - 1, 1, osem1)

    out5 = emb_kernel(tokens_t, weight)
    out = out5.transpose(2, 4, 0, 1, 3).reshape(batch, hist, EMB)
    return out


# speedup vs baseline: 1.0006x; 1.0006x over previous
import functools
import math

import jax
import jax.numpy as jnp
from jax import lax
from jax.experimental import pallas as pl
from jax.experimental.pallas import tpu as pltpu
from jax.experimental.pallas import tpu_sc as plsc

EMB = 64
NC = 2
NS = 16
NW = NC * NS
LANES = 16
CHUNK = 128
SCALE = math.sqrt(EMB)


def kernel(tokens, weight):
    batch, hist = tokens.shape
    b_per_w = batch // NW
    n_bchunk = b_per_w // CHUNK
    assert b_per_w * NW == batch and n_bchunk * CHUNK == b_per_w
    eb = EMB // 8
    bb = batch // CHUNK

    tokens_t = tokens.T.astype(jnp.int32)

    mesh = plsc.VectorSubcoreMesh(core_axis_name="c", subcore_axis_name="s")

    stage_sz = eb * n_bchunk * 8 * CHUNK
    espan = n_bchunk * 8 * CHUNK

    @functools.partial(
        pl.kernel,
        mesh=mesh,
        out_type=jax.ShapeDtypeStruct((hist * eb * bb * 8 * CHUNK,),
                                      jnp.float32),
        scratch_types=[
            pltpu.VMEM((hist, b_per_w), jnp.int32),
            pltpu.VMEM((n_bchunk, CHUNK, EMB), jnp.float32),
            pltpu.VMEM((2, stage_sz), jnp.float32),
            pltpu.SemaphoreType.DMA((n_bchunk,)),
            pltpu.SemaphoreType.DMA,
            pltpu.SemaphoreType.DMA,
        ],
        compiler_params=pltpu.CompilerParams(use_tc_tiling_on_sc=False,
                                             needs_layout_passes=False),
    )
    def emb_kernel(tok_hbm, table_hbm, out_hbm, tok_v, rows_v, stage_v,
                   gsem, osem0, osem1):
        wid = lax.axis_index("s") * NC + lax.axis_index("c")
        base_b = wid * b_per_w
        pltpu.sync_copy(tok_hbm.at[:, pl.ds(base_b, b_per_w)], tok_v)

        iota = lax.iota(jnp.int32, LANES)
        addr_m = [(iota // 8 + 2 * m) * espan + (iota % 8) * CHUNK
                  for m in range(EMB // LANES)]

        def gather(h, c):
            pltpu.async_copy(
                table_hbm.at[tok_v.at[h, pl.ds(CHUNK * c, CHUNK)]],
                rows_v.at[c], gsem.at[c])

        def wait_gather(h, c):
            pltpu.make_async_copy(
                table_hbm.at[tok_v.at[h, pl.ds(CHUNK * c, CHUNK)]],
                rows_v.at[c], gsem.at[c]).wait()

        def out_pieces(h, par):
            base = h * (eb * bb * 8 * CHUNK) + wid * (n_bchunk * 8 * CHUNK)
            for e0 in range(eb):
                yield (stage_v.at[par, pl.ds(e0 * espan, espan)],
                       out_hbm.at[pl.ds(base + e0 * (bb * 8 * CHUNK), espan)])

        def write_out(h, par, sem):
            for src, dst in out_pieces(h, par):
                pltpu.async_copy(src, dst, sem)

        def wait_out(h, par, sem):
            for src, dst in out_pieces(h, par):
                pltpu.make_async_copy(src, dst, sem).wait()

        def transpose_chunk(h2, c):
            src = rows_v.at[c]
            dst = stage_v.at[h2]
            coff = c * (8 * CHUNK)

            @plsc.parallel_loop(0, CHUNK, unroll=4)
            def rbody(r):
                off = lax.broadcast(coff + r, (LANES,))
                for m in range(EMB // LANES):
                    vec = src[r, pl.ds(LANES * m, LANES)]
                    plsc.store_scatter(dst, [addr_m[m] + off], vec * SCALE)

        for c in range(n_bchunk):
            gather(0, c)

        def hbody(h, carry):
            h2 = h % 2
            even = h2 == 0

            @pl.when((h >= 2) & even)
            def _():
                wait_out(h, 0, osem0)

            @pl.when((h >= 2) & jnp.logical_not(even))
            def _():
                wait_out(h, 1, osem1)

            for c in range(n_bchunk):
                wait_gather(h, c)
                transpose_chunk(h2, c)

                @pl.when(h < hist - 1)
                def _():
                    gather(h + 1, c)

            @pl.when(even)
            def _():
                write_out(h, 0, osem0)

            @pl.when(jnp.logical_not(even))
            def _():
                write_out(h, 1, osem1)

            return carry

        lax.fori_loop(0, hist, hbody, 0)
        wait_out(hist - 2, 0, osem0)
        wait_out(hist - 1, 1, osem1)

    out5 = emb_kernel(tokens_t, weight).reshape(hist, eb, bb, 8, CHUNK)
    out = out5.transpose(2, 4, 0, 1, 3).reshape(batch, hist, EMB)
    return out


# speedup vs baseline: 1.0546x; 1.0540x over previous
import functools
import math

import jax
import jax.numpy as jnp
from jax import lax
from jax.experimental import pallas as pl
from jax.experimental.pallas import tpu as pltpu
from jax.experimental.pallas import tpu_sc as plsc

EMB = 64
NC = 2
NS = 16
NW = NC * NS
LANES = 16
CHUNK = 128
SCALE = math.sqrt(EMB)


def kernel(tokens, weight):
    batch, hist = tokens.shape
    b_per_w = batch // NW
    n_bchunk = b_per_w // CHUNK
    assert b_per_w * NW == batch and n_bchunk * CHUNK == b_per_w
    eb = EMB // 8
    bb = batch // CHUNK

    tokens_t = tokens.T.astype(jnp.int32)

    mesh = plsc.VectorSubcoreMesh(core_axis_name="c", subcore_axis_name="s")

    stage_sz = eb * n_bchunk * 8 * CHUNK
    espan = n_bchunk * 8 * CHUNK

    @functools.partial(
        pl.kernel,
        mesh=mesh,
        out_type=jax.ShapeDtypeStruct((hist * eb * bb * 8 * CHUNK,),
                                      jnp.float32),
        scratch_types=[
            pltpu.VMEM((hist, b_per_w), jnp.int32),
            pltpu.VMEM((2, CHUNK, 2 * EMB), jnp.float32),
            pltpu.VMEM((2, stage_sz), jnp.float32),
            pltpu.SemaphoreType.DMA((2,)),
            pltpu.SemaphoreType.DMA,
            pltpu.SemaphoreType.DMA,
        ],
        compiler_params=pltpu.CompilerParams(use_tc_tiling_on_sc=False,
                                             needs_layout_passes=False),
    )
    def emb_kernel(tok_hbm, table_hbm, out_hbm, tok_v, rows_v, stage_v,
                   gsem, osem0, osem1):
        wid = lax.axis_index("s") * NC + lax.axis_index("c")
        base_b = wid * b_per_w
        pltpu.sync_copy(tok_hbm.at[:, pl.ds(base_b, b_per_w)], tok_v)

        iota = lax.iota(jnp.int32, LANES)
        addr_m = [(iota // 8 + 2 * m) * espan + (iota % 8) * CHUNK
                  for m in range(EMB // LANES)]

        def gather(h, c):
            pltpu.async_copy(
                table_hbm.at[tok_v.at[h, pl.ds(CHUNK * c, CHUNK)]],
                rows_v.at[c % 2], gsem.at[c % 2])

        def wait_gather(h, c):
            pltpu.make_async_copy(
                table_hbm.at[tok_v.at[h, pl.ds(CHUNK * c, CHUNK)]],
                rows_v.at[c % 2], gsem.at[c % 2]).wait()

        def out_pieces(h, par):
            base = h * (eb * bb * 8 * CHUNK) + wid * (n_bchunk * 8 * CHUNK)
            for e0 in range(eb):
                yield (stage_v.at[par, pl.ds(e0 * espan, espan)],
                       out_hbm.at[pl.ds(base + e0 * (bb * 8 * CHUNK), espan)])

        def write_out(h, par, sem):
            for src, dst in out_pieces(h, par):
                pltpu.async_copy(src, dst, sem)

        def wait_out(h, par, sem):
            for src, dst in out_pieces(h, par):
                pltpu.make_async_copy(src, dst, sem).wait()

        def transpose_chunk(h2, c):
            src = rows_v.at[c % 2]
            dst = stage_v.at[h2]
            coff = c * (8 * CHUNK)

            @plsc.parallel_loop(0, CHUNK, unroll=4)
            def rbody(r):
                off = lax.broadcast(coff + r, (LANES,))
                for m in range(EMB // LANES):
                    vec = src[r, pl.ds(LANES * m, LANES)]
                    plsc.store_scatter(dst, [addr_m[m] + off], vec * SCALE)

        gather(0, 0)
        gather(0, 1)

        def hbody(h, carry):
            h2 = h % 2
            even = h2 == 0

            @pl.when((h >= 2) & even)
            def _():
                wait_out(h, 0, osem0)

            @pl.when((h >= 2) & jnp.logical_not(even))
            def _():
                wait_out(h, 1, osem1)

            for c in range(n_bchunk):
                wait_gather(h, c)
                transpose_chunk(h2, c)
                if c < 2:
                    gather(h, c + 2)
                else:
                    @pl.when(h < hist - 1)
                    def _():
                        gather(h + 1, c - 2)

            @pl.when(even)
            def _():
                write_out(h, 0, osem0)

            @pl.when(jnp.logical_not(even))
            def _():
                write_out(h, 1, osem1)

            return carry

        lax.fori_loop(0, hist, hbody, 0)
        wait_out(hist - 2, 0, osem0)
        wait_out(hist - 1, 1, osem1)

    wp = jnp.pad(weight, ((0, 0), (0, EMB)))
    out5 = emb_kernel(tokens_t, wp).reshape(hist, eb, bb, 8, CHUNK)
    out = out5.transpose(2, 4, 0, 1, 3).reshape(batch, hist, EMB)
    return out
